# Initial kernel scaffold; baseline (speedup 1.0000x reference)
import jax
import jax.numpy as jnp
from jax.experimental import pallas as pl


def kernel(x, edge_index, edge_weight, Wcz, bcz, Wcr, bcr, Wch, bch, Wz, bz, Wr, br, Wh, bh, Wl, bl):
    raise NotImplementedError("write your pallas kernel here")


# speedup vs baseline: 18.3642x; 18.3642x over previous
import jax
import jax.numpy as jnp
from jax import lax
from jax.experimental import pallas as pl
from jax.experimental.pallas import tpu as pltpu
from jax.experimental.pallas import tpu_sc as plsc

_NC = 2
_NS = 16
_L = 16
_CH = 128


def _fold_matmul_body(x_ref, wcz_ref, wch_ref, wz_ref, wh_ref, q_ref):
    fh = wcz_ref.shape[1]
    wf = jnp.concatenate(
        [wcz_ref[...] @ wz_ref[:fh, :], wch_ref[...] @ wh_ref[:fh, :]], axis=1
    )
    q_ref[...] = x_ref[...] @ wf


def _tail_body(agg_ref, bcz_ref, bch_ref, wz_ref, bz_ref, wh_ref, bh_ref,
               wl_ref, bl_ref, out_ref):
    fh = bz_ref.shape[0]
    agg = agg_ref[0] + agg_ref[1]
    bze = bcz_ref[...] @ wz_ref[:fh, :] + bz_ref[...]
    bhe = bch_ref[...] @ wh_ref[:fh, :] + bh_ref[...]
    z = jax.nn.sigmoid(agg[:, :fh] + bze)
    ht = jnp.tanh(agg[:, fh:] + bhe)
    h = jax.nn.relu((1.0 - z) * ht)
    g = h @ wl_ref[...] + bl_ref[...]
    out_ref[...] = jax.nn.softmax(g, axis=-1)


def _sc_body(q_hbm, src_hbm, dst_hbm, w_hbm, z2_hbm, z1_hbm, agg_hbm,
             idx_s, idx_d, wbuf, nbuf, rows, dinv, deg_sh, agg_sh, sem):
    c = lax.axis_index("c")
    s = lax.axis_index("s")

    n = dinv.shape[0]
    fw = rows.shape[1]
    zrows = (n // (8 * _NS)) * 8
    zrem = n - zrows * _NS
    ep = w_hbm.shape[0]
    deg_chunks = ep // (_NS * _CH)
    w_chunks = ep // (_NC * _NS * _CH)

    @pl.when(s == 0)
    def _():
        pltpu.sync_copy(z1_hbm, deg_sh)

    pltpu.sync_copy(z2_hbm.at[pl.ds(s * zrows, zrows)],
                    agg_sh.at[pl.ds(s * zrows, zrows)])
    if zrem:
        @pl.when(s == _NS - 1)
        def _():
            pltpu.sync_copy(z2_hbm.at[pl.ds(_NS * zrows, zrem)],
                            agg_sh.at[pl.ds(_NS * zrows, zrem)])
    plsc.subcore_barrier()

    def deg_step(i, carry):
        base = (s * deg_chunks + i) * _CH
        pltpu.sync_copy(dst_hbm.at[pl.ds(base, _CH)], idx_d)
        pltpu.sync_copy(w_hbm.at[pl.ds(base, _CH)], wbuf)
        pltpu.sync_copy(wbuf, deg_sh.at[idx_d], add=True)
        return carry

    lax.fori_loop(0, deg_chunks, deg_step, 0)
    plsc.subcore_barrier()

    pltpu.sync_copy(deg_sh, dinv)

    def rsqrt_step(i, carry):
        xv = dinv[pl.ds(i * _L, _L)]
        ib = lax.bitcast_convert_type(xv, jnp.int32)
        y = lax.bitcast_convert_type(jnp.int32(0x5F3759DF) - (ib >> 1), jnp.float32)
        y = y * (1.5 - 0.5 * xv * y * y)
        y = y * (1.5 - 0.5 * xv * y * y)
        y = y * (1.5 - 0.5 * xv * y * y)
        dinv[pl.ds(i * _L, _L)] = y
        return carry

    lax.fori_loop(0, n // _L, rsqrt_step, 0)

    wid = c * _NS + s

    def main_step(i, carry):
        base = (wid * w_chunks + i) * _CH
        pltpu.sync_copy(src_hbm.at[pl.ds(base, _CH)], idx_s)
        pltpu.sync_copy(dst_hbm.at[pl.ds(base, _CH)], idx_d)
        pltpu.sync_copy(w_hbm.at[pl.ds(base, _CH)], wbuf)
        pltpu.async_copy(q_hbm.at[idx_s], rows, sem).wait()

        def norm_step(j, cc):
            sv = idx_s[pl.ds(j * _L, _L)]
            dv = idx_d[pl.ds(j * _L, _L)]
            nb = (plsc.load_gather(dinv, [sv]) * wbuf[pl.ds(j * _L, _L)]
                  * plsc.load_gather(dinv, [dv]))
            nbuf[pl.ds(j * _L, _L)] = nb
            return cc

        lax.fori_loop(0, _CH // _L, norm_step, 0)

        def scale_step(g, cc):
            n16 = nbuf[pl.ds(g * _L, _L)]
            for k in range(_L):
                ns = n16[k]
                e = g * _L + k
                for j in range(fw // _L):
                    rows[e, pl.ds(j * _L, _L)] = rows[e, pl.ds(j * _L, _L)] * ns
            return cc

        lax.fori_loop(0, _CH // _L, scale_step, 0)
        pltpu.sync_copy(rows, agg_sh.at[idx_d], add=True)
        return carry

    lax.fori_loop(0, w_chunks, main_step, 0)
    plsc.subcore_barrier()

    pltpu.sync_copy(agg_sh.at[pl.ds(s * zrows, zrows)],
                    agg_hbm.at[c, pl.ds(s * zrows, zrows)])
    if zrem:
        @pl.when(s == _NS - 1)
        def _():
            pltpu.sync_copy(agg_sh.at[pl.ds(_NS * zrows, zrem)],
                            agg_hbm.at[c, pl.ds(_NS * zrows, zrem)])


def kernel(x, edge_index, edge_weight, Wcz, bcz, Wcr, bcr, Wch, bch,
           Wz, bz, Wr, br, Wh, bh, Wl, bl):
    n, _ = x.shape
    fh = Wcz.shape[1]
    fw = 2 * fh
    e = edge_weight.shape[0]

    q = pl.pallas_call(
        _fold_matmul_body,
        out_shape=jax.ShapeDtypeStruct((n, fw), jnp.float32),
    )(x, Wcz, Wch, Wz, Wh)

    loop = jnp.arange(n, dtype=edge_index.dtype)
    src = jnp.concatenate([edge_index[0], loop])
    dst = jnp.concatenate([edge_index[1], loop])
    w = jnp.concatenate([edge_weight, jnp.ones((n,), edge_weight.dtype)])
    grp = _NC * _NS * _CH
    ep = ((e + n + grp - 1) // grp) * grp
    pad = ep - (e + n)
    src = jnp.concatenate([src, jnp.zeros((pad,), src.dtype)])
    dst = jnp.concatenate([dst, jnp.zeros((pad,), dst.dtype)])
    w = jnp.concatenate([w, jnp.zeros((pad,), w.dtype)])

    z2 = jnp.zeros((n, fw), jnp.float32)
    z1 = jnp.zeros((n,), jnp.float32)

    mesh = plsc.VectorSubcoreMesh(core_axis_name="c", subcore_axis_name="s")
    agg2 = pl.kernel(
        _sc_body,
        out_type=jax.ShapeDtypeStruct((_NC, n, fw), jnp.float32),
        mesh=mesh,
        compiler_params=pltpu.CompilerParams(needs_layout_passes=False),
        scratch_types=[
            pltpu.VMEM((_CH,), jnp.int32),
            pltpu.VMEM((_CH,), jnp.int32),
            pltpu.VMEM((_CH,), jnp.float32),
            pltpu.VMEM((_CH,), jnp.float32),
            pltpu.VMEM((_CH, fw), jnp.float32),
            pltpu.VMEM((n,), jnp.float32),
            pltpu.VMEM_SHARED((n,), jnp.float32),
            pltpu.VMEM_SHARED((n, fw), jnp.float32),
            pltpu.SemaphoreType.DMA,
        ],
    )(q, src, dst, w, z2, z1)

    return pl.pallas_call(
        _tail_body,
        out_shape=jax.ShapeDtypeStruct((n, Wl.shape[1]), jnp.float32),
    )(agg2, bcz, bch, Wz, bz, Wh, bh, Wl, bl)


# speedup vs baseline: 25.6676x; 1.3977x over previous
import jax
import jax.numpy as jnp
from jax import lax
from jax.experimental import pallas as pl
from jax.experimental.pallas import tpu as pltpu
from jax.experimental.pallas import tpu_sc as plsc

_NC = 2
_NS = 16
_L = 16
_CH = 128


def _fold_matmul_body(x_ref, wcz_ref, wch_ref, wz_ref, wh_ref, q_ref):
    fh = wcz_ref.shape[1]
    q_ref[:, :fh] = x_ref[...] @ (wcz_ref[...] @ wz_ref[:fh, :])
    q_ref[:, fh:] = x_ref[...] @ (wch_ref[...] @ wh_ref[:fh, :])


def _tail_body(agg_ref, bcz_ref, bch_ref, wz_ref, bz_ref, wh_ref,
               bh_ref, wl_ref, bl_ref, out_ref):
    fh = bz_ref.shape[0]
    p = agg_ref[0] + agg_ref[1]
    bze = bcz_ref[...] @ wz_ref[:fh, :] + bz_ref[...]
    bhe = bch_ref[...] @ wh_ref[:fh, :] + bh_ref[...]
    z = jax.nn.sigmoid(p[:, :fh] + bze)
    ht = jnp.tanh(p[:, fh:] + bhe)
    h = jax.nn.relu((1.0 - z) * ht)
    g = h @ wl_ref[...] + bl_ref[...]
    out_ref[...] = jax.nn.softmax(g, axis=-1)


def _sc_body(q_hbm, edata_hbm, z2_hbm, z1_hbm, agg_hbm,
             ed, wf32, nbuf, rows,
             dinv, deg_sh, agg_sh,
             esem, dsem, rsem):
    c = lax.axis_index("c")
    s = lax.axis_index("s")

    n = dinv.shape[0]
    fw = rows.shape[1]
    chunks = edata_hbm.shape[1]
    half = chunks // 2
    zrows = (n // (8 * _NS)) * 8
    zrem = n - zrows * _NS

    def load_echunk(k):
        pltpu.async_copy(edata_hbm.at[s, k], ed, esem).wait()

    @pl.when(s == 0)
    def _():
        pltpu.sync_copy(z1_hbm, deg_sh)

    pltpu.sync_copy(z2_hbm.at[pl.ds(s * zrows, zrows)],
                    agg_sh.at[pl.ds(s * zrows, zrows)])
    if zrem:
        @pl.when(s == _NS - 1)
        def _():
            pltpu.sync_copy(z2_hbm.at[pl.ds(_NS * zrows, zrem)],
                            agg_sh.at[pl.ds(_NS * zrows, zrem)])
    plsc.subcore_barrier()

    def deg_step(j, carry):
        load_echunk(j)
        for k in range(_CH // _L):
            wf32[0, pl.ds(k * _L, _L)] = lax.bitcast_convert_type(
                ed[2, pl.ds(k * _L, _L)], jnp.float32)
        pltpu.async_copy(wf32.at[0], deg_sh.at[ed.at[1]],
                         dsem, add=True).wait()
        return carry

    lax.fori_loop(0, chunks, deg_step, 0)
    plsc.subcore_barrier()

    pltpu.sync_copy(deg_sh, dinv)

    def rsqrt_step(i, carry):
        xv = dinv[pl.ds(i * _L, _L)]
        ib = lax.bitcast_convert_type(xv, jnp.int32)
        y = lax.bitcast_convert_type(jnp.int32(0x5F3759DF) - (ib >> 1),
                                     jnp.float32)
        y = y * (1.5 - 0.5 * xv * y * y)
        y = y * (1.5 - 0.5 * xv * y * y)
        y = y * (1.5 - 0.5 * xv * y * y)
        dinv[pl.ds(i * _L, _L)] = y
        return carry

    lax.fori_loop(0, n // _L, rsqrt_step, 0)

    def main_step(i, carry):
        load_echunk(c * half + i)

        gcp = pltpu.async_copy(q_hbm.at[ed.at[0]], rows, rsem)

        for j in range(_CH // _L):
            sv = ed[0, pl.ds(j * _L, _L)]
            dv = ed[1, pl.ds(j * _L, _L)]
            wv = lax.bitcast_convert_type(
                ed[2, pl.ds(j * _L, _L)], jnp.float32)
            nbuf[pl.ds(j * _L, _L)] = (plsc.load_gather(dinv, [sv]) * wv
                                       * plsc.load_gather(dinv, [dv]))

        gcp.wait()

        def scale_step(g2, cc):
            n16 = nbuf[pl.ds(g2 * _L, _L)]
            for k in range(_L):
                ns = n16[k]
                e = g2 * _L + k
                for j in range(fw // _L):
                    rows[e, pl.ds(j * _L, _L)] = (
                        rows[e, pl.ds(j * _L, _L)] * ns)
            return cc

        lax.fori_loop(0, _CH // _L, scale_step, 0)

        pltpu.async_copy(rows, agg_sh.at[ed.at[1]], rsem, add=True).wait()
        return carry

    lax.fori_loop(0, half, main_step, 0)
    plsc.subcore_barrier()

    pltpu.sync_copy(agg_sh.at[pl.ds(s * zrows, zrows)],
                    agg_hbm.at[c, pl.ds(s * zrows, zrows)])
    if zrem:
        @pl.when(s == _NS - 1)
        def _():
            pltpu.sync_copy(agg_sh.at[pl.ds(_NS * zrows, zrem)],
                            agg_hbm.at[c, pl.ds(_NS * zrows, zrem)])


def kernel(x, edge_index, edge_weight, Wcz, bcz, Wcr, bcr, Wch, bch,
           Wz, bz, Wr, br, Wh, bh, Wl, bl):
    n, _ = x.shape
    fh = Wcz.shape[1]
    e = edge_weight.shape[0]

    q = pl.pallas_call(
        _fold_matmul_body,
        out_shape=jax.ShapeDtypeStruct((n, 2 * fh), jnp.float32),
    )(x, Wcz, Wch, Wz, Wh)

    loop = jnp.arange(n, dtype=edge_index.dtype)
    src = jnp.concatenate([edge_index[0], loop])
    dst = jnp.concatenate([edge_index[1], loop])
    w = jnp.concatenate([edge_weight, jnp.ones((n,), edge_weight.dtype)])
    grp = _NS * _CH * 4
    ep = ((e + n + grp - 1) // grp) * grp
    pad = ep - (e + n)
    padv = jnp.arange(pad, dtype=src.dtype) % n
    src = jnp.concatenate([src, padv])
    dst = jnp.concatenate([dst, padv])
    w = jnp.concatenate([w, jnp.zeros((pad,), w.dtype)])
    chunks = ep // (_NS * _CH)
    edata = jnp.stack(
        [src.reshape(-1, _CH), dst.reshape(-1, _CH),
         lax.bitcast_convert_type(w, jnp.int32).reshape(-1, _CH)], axis=1
    ).reshape(_NS, chunks, 3, _CH)

    z2 = jnp.zeros((n, 2 * fh), jnp.float32)
    z1 = jnp.zeros((n,), jnp.float32)

    mesh = plsc.VectorSubcoreMesh(core_axis_name="c", subcore_axis_name="s")
    agg2 = pl.kernel(
        _sc_body,
        out_type=jax.ShapeDtypeStruct((_NC, n, 2 * fh), jnp.float32),
        mesh=mesh,
        compiler_params=pltpu.CompilerParams(needs_layout_passes=False),
        scratch_types=[
            pltpu.VMEM((3, _CH), jnp.int32),
            pltpu.VMEM((1, _CH), jnp.float32),
            pltpu.VMEM((_CH,), jnp.float32),
            pltpu.VMEM((_CH, 2 * fh), jnp.float32),
            pltpu.VMEM((n,), jnp.float32),
            pltpu.VMEM_SHARED((n,), jnp.float32),
            pltpu.VMEM_SHARED((n, 2 * fh), jnp.float32),
            pltpu.SemaphoreType.DMA,
            pltpu.SemaphoreType.DMA,
            pltpu.SemaphoreType.DMA,
        ],
    )(q, edata, z2, z1)

    return pl.pallas_call(
        _tail_body,
        out_shape=jax.ShapeDtypeStruct((n, Wl.shape[1]), jnp.float32),
    )(agg2, bcz, bch, Wz, bz, Wh, bh, Wl, bl)


# speedup vs baseline: 29.1440x; 1.1354x over previous
import jax
import jax.numpy as jnp
from jax import lax
from jax.experimental import pallas as pl
from jax.experimental.pallas import tpu as pltpu
from jax.experimental.pallas import tpu_sc as plsc

_NC = 2
_NS = 16
_L = 16
_CH = 128


def _fold_matmul_body(x_ref, wcz_ref, wch_ref, wz_ref, wh_ref, q_ref):
    fh = wcz_ref.shape[1]
    q_ref[:, :fh] = x_ref[...] @ (wcz_ref[...] @ wz_ref[:fh, :])
    q_ref[:, fh:] = x_ref[...] @ (wch_ref[...] @ wh_ref[:fh, :])


def _tail_body(agg_ref, bcz_ref, bch_ref, wz_ref, bz_ref, wh_ref,
               bh_ref, wl_ref, bl_ref, out_ref):
    fh = bz_ref.shape[0]
    p = agg_ref[0] + agg_ref[1]
    bze = bcz_ref[...] @ wz_ref[:fh, :] + bz_ref[...]
    bhe = bch_ref[...] @ wh_ref[:fh, :] + bh_ref[...]
    z = jax.nn.sigmoid(p[:, :fh] + bze)
    ht = jnp.tanh(p[:, fh:] + bhe)
    h = jax.nn.relu((1.0 - z) * ht)
    g = h @ wl_ref[...] + bl_ref[...]
    out_ref[...] = jax.nn.softmax(g, axis=-1)


def _sc_body(q_hbm, edata_hbm, z2_hbm, z1_hbm, agg_hbm,
             ed0, ed1, wf32, nbuf, rows,
             dinv, deg_sh, agg_sh,
             esem, dsem, rsem):
    ed = (ed0, ed1)
    c = lax.axis_index("c")
    s = lax.axis_index("s")

    n = dinv.shape[0]
    fw = rows.shape[1]
    chunks = edata_hbm.shape[1]
    half = chunks // 2
    zrows = (n // (8 * _NS)) * 8
    zrem = n - zrows * _NS

    def start_echunk(k, slot):
        pltpu.async_copy(edata_hbm.at[s, k], ed[slot], esem)

    def wait_echunk(slot):
        pltpu.make_async_copy(edata_hbm.at[s, 0], ed[slot], esem).wait()

    start_echunk(0, 0)
    @pl.when(s == 0)
    def _():
        pltpu.sync_copy(z1_hbm, deg_sh)

    pltpu.sync_copy(z2_hbm.at[pl.ds(s * zrows, zrows)],
                    agg_sh.at[pl.ds(s * zrows, zrows)])
    if zrem:
        @pl.when(s == _NS - 1)
        def _():
            pltpu.sync_copy(z2_hbm.at[pl.ds(_NS * zrows, zrem)],
                            agg_sh.at[pl.ds(_NS * zrows, zrem)])
    plsc.subcore_barrier()

    def deg_group(gq, carry):
        for slot in range(2):
            j = gq * 2 + slot
            wait_echunk(slot)

            for k in range(_CH // _L):
                wf32[slot, pl.ds(k * _L, _L)] = lax.bitcast_convert_type(
                    ed[slot][2, pl.ds(k * _L, _L)], jnp.float32)

            @pl.when(j >= 1)
            def _():
                pltpu.make_async_copy(wf32.at[0], deg_sh.at[ed[0].at[1]],
                                      dsem).wait()

            @pl.when(j + 1 < chunks)
            def _():
                start_echunk(j + 1, 1 - slot)

            pltpu.async_copy(wf32.at[slot], deg_sh.at[ed[slot].at[1]],
                             dsem, add=True)
        return carry

    lax.fori_loop(0, chunks // 2, deg_group, 0)
    pltpu.make_async_copy(wf32.at[0], deg_sh.at[ed[0].at[1]], dsem).wait()
    plsc.subcore_barrier()

    pltpu.sync_copy(deg_sh, dinv)

    def rsqrt_step(i, carry):
        xv = dinv[pl.ds(i * _L, _L)]
        ib = lax.bitcast_convert_type(xv, jnp.int32)
        y = lax.bitcast_convert_type(jnp.int32(0x5F3759DF) - (ib >> 1),
                                     jnp.float32)
        y = y * (1.5 - 0.5 * xv * y * y)
        y = y * (1.5 - 0.5 * xv * y * y)
        y = y * (1.5 - 0.5 * xv * y * y)
        dinv[pl.ds(i * _L, _L)] = y
        return carry

    lax.fori_loop(0, n // _L, rsqrt_step, 0)

    start_echunk(c * half, 0)

    def main_group(gq, carry):
        for slot in range(2):
            i = gq * 2 + slot
            wait_echunk(slot)

            @pl.when(i + 1 < half)
            def _():
                start_echunk(c * half + i + 1, 1 - slot)

            gcp = pltpu.async_copy(q_hbm.at[ed[slot].at[0]], rows, rsem)

            for j in range(_CH // _L):
                sv = ed[slot][0, pl.ds(j * _L, _L)]
                dv = ed[slot][1, pl.ds(j * _L, _L)]
                wv = lax.bitcast_convert_type(
                    ed[slot][2, pl.ds(j * _L, _L)], jnp.float32)
                nbuf[pl.ds(j * _L, _L)] = (plsc.load_gather(dinv, [sv]) * wv
                                           * plsc.load_gather(dinv, [dv]))

            gcp.wait()

            def scale_step(g2, cc):
                n16 = nbuf[pl.ds(g2 * _L, _L)]
                for k in range(_L):
                    ns = n16[k]
                    e = g2 * _L + k
                    for j in range(fw // _L):
                        rows[e, pl.ds(j * _L, _L)] = (
                            rows[e, pl.ds(j * _L, _L)] * ns)
                return cc

            lax.fori_loop(0, _CH // _L, scale_step, 0)

            pltpu.async_copy(rows, agg_sh.at[ed[slot].at[1]],
                             rsem, add=True).wait()
        return carry

    lax.fori_loop(0, half // 2, main_group, 0)
    plsc.subcore_barrier()

    pltpu.sync_copy(agg_sh.at[pl.ds(s * zrows, zrows)],
                    agg_hbm.at[c, pl.ds(s * zrows, zrows)])
    if zrem:
        @pl.when(s == _NS - 1)
        def _():
            pltpu.sync_copy(agg_sh.at[pl.ds(_NS * zrows, zrem)],
                            agg_hbm.at[c, pl.ds(_NS * zrows, zrem)])


def kernel(x, edge_index, edge_weight, Wcz, bcz, Wcr, bcr, Wch, bch,
           Wz, bz, Wr, br, Wh, bh, Wl, bl):
    n, _ = x.shape
    fh = Wcz.shape[1]
    e = edge_weight.shape[0]

    q = pl.pallas_call(
        _fold_matmul_body,
        out_shape=jax.ShapeDtypeStruct((n, 2 * fh), jnp.float32),
    )(x, Wcz, Wch, Wz, Wh)

    loop = jnp.arange(n, dtype=edge_index.dtype)
    src = jnp.concatenate([edge_index[0], loop])
    dst = jnp.concatenate([edge_index[1], loop])
    w = jnp.concatenate([edge_weight, jnp.ones((n,), edge_weight.dtype)])
    grp = _NS * _CH * 4
    ep = ((e + n + grp - 1) // grp) * grp
    pad = ep - (e + n)
    padv = jnp.arange(pad, dtype=src.dtype) % n
    src = jnp.concatenate([src, padv])
    dst = jnp.concatenate([dst, padv])
    w = jnp.concatenate([w, jnp.zeros((pad,), w.dtype)])
    chunks = ep // (_NS * _CH)
    edata = jnp.stack(
        [src.reshape(-1, _CH), dst.reshape(-1, _CH),
         lax.bitcast_convert_type(w, jnp.int32).reshape(-1, _CH)], axis=1
    ).reshape(_NS, chunks, 3, _CH)

    z2 = jnp.zeros((n, 2 * fh), jnp.float32)
    z1 = jnp.zeros((n,), jnp.float32)

    mesh = plsc.VectorSubcoreMesh(core_axis_name="c", subcore_axis_name="s")
    agg2 = pl.kernel(
        _sc_body,
        out_type=jax.ShapeDtypeStruct((_NC, n, 2 * fh), jnp.float32),
        mesh=mesh,
        compiler_params=pltpu.CompilerParams(needs_layout_passes=False),
        scratch_types=[
            pltpu.VMEM((3, _CH), jnp.int32),
            pltpu.VMEM((3, _CH), jnp.int32),
            pltpu.VMEM((2, _CH), jnp.float32),
            pltpu.VMEM((_CH,), jnp.float32),
            pltpu.VMEM((_CH, 2 * fh), jnp.float32),
            pltpu.VMEM((n,), jnp.float32),
            pltpu.VMEM_SHARED((n,), jnp.float32),
            pltpu.VMEM_SHARED((n, 2 * fh), jnp.float32),
            pltpu.SemaphoreType.DMA,
            pltpu.SemaphoreType.DMA,
            pltpu.SemaphoreType.DMA,
        ],
    )(q, edata, z2, z1)

    return pl.pallas_call(
        _tail_body,
        out_shape=jax.ShapeDtypeStruct((n, Wl.shape[1]), jnp.float32),
    )(agg2, bcz, bch, Wz, bz, Wh, bh, Wl, bl)


# speedup vs baseline: 33.1082x; 1.1360x over previous
import jax
import jax.numpy as jnp
from jax import lax
from jax.experimental import pallas as pl
from jax.experimental.pallas import tpu as pltpu
from jax.experimental.pallas import tpu_sc as plsc

_NC = 2
_NS = 16
_L = 16
_CH = 128


def _fold_matmul_body(x_ref, wcz_ref, wch_ref, wz_ref, wh_ref, q_ref):
    fh = wcz_ref.shape[1]
    q_ref[:, :fh] = x_ref[...] @ (wcz_ref[...] @ wz_ref[:fh, :])
    q_ref[:, fh:] = x_ref[...] @ (wch_ref[...] @ wh_ref[:fh, :])


def _tail_body(agg_ref, bcz_ref, bch_ref, wz_ref, bz_ref, wh_ref,
               bh_ref, wl_ref, bl_ref, out_ref):
    fh = bz_ref.shape[0]
    p = agg_ref[0] + agg_ref[1]
    bze = bcz_ref[...] @ wz_ref[:fh, :] + bz_ref[...]
    bhe = bch_ref[...] @ wh_ref[:fh, :] + bh_ref[...]
    z = jax.nn.sigmoid(p[:, :fh] + bze)
    ht = jnp.tanh(p[:, fh:] + bhe)
    h = jax.nn.relu((1.0 - z) * ht)
    g = h @ wl_ref[...] + bl_ref[...]
    out_ref[...] = jax.nn.softmax(g, axis=-1)


def _sc_body(q_hbm, edata_hbm, z2_hbm, z1_hbm, agg_hbm,
             ed0, ed1, wf32, nbuf, rows0, rows1,
             dinv, deg_sh, agg_sh,
             esem, dsem, rsem, ssem):
    ed = (ed0, ed1)
    rows = (rows0, rows1)
    c = lax.axis_index("c")
    s = lax.axis_index("s")

    n = dinv.shape[0]
    fw = rows0.shape[1]
    chunks = edata_hbm.shape[1]
    half = chunks // 2
    zrows = (n // (8 * _NS)) * 8
    zrem = n - zrows * _NS

    def start_echunk(k, slot):
        pltpu.async_copy(edata_hbm.at[s, k], ed[slot], esem)

    def wait_echunk(slot):
        pltpu.make_async_copy(edata_hbm.at[s, 0], ed[slot], esem).wait()

    start_echunk(0, 0)
    @pl.when(s == 0)
    def _():
        pltpu.sync_copy(z1_hbm, deg_sh)

    pltpu.sync_copy(z2_hbm.at[pl.ds(s * zrows, zrows)],
                    agg_sh.at[pl.ds(s * zrows, zrows)])
    if zrem:
        @pl.when(s == _NS - 1)
        def _():
            pltpu.sync_copy(z2_hbm.at[pl.ds(_NS * zrows, zrem)],
                            agg_sh.at[pl.ds(_NS * zrows, zrem)])
    plsc.subcore_barrier()

    def deg_group(gq, carry):
        for slot in range(2):
            j = gq * 2 + slot
            wait_echunk(slot)

            for k in range(_CH // _L):
                wf32[slot, pl.ds(k * _L, _L)] = lax.bitcast_convert_type(
                    ed[slot][2, pl.ds(k * _L, _L)], jnp.float32)

            @pl.when(j >= 1)
            def _():
                pltpu.make_async_copy(wf32.at[0], deg_sh.at[ed[0].at[1]],
                                      dsem).wait()

            @pl.when(j + 1 < chunks)
            def _():
                start_echunk(j + 1, 1 - slot)

            pltpu.async_copy(wf32.at[slot], deg_sh.at[ed[slot].at[1]],
                             dsem, add=True)
        return carry

    lax.fori_loop(0, chunks // 2, deg_group, 0)
    pltpu.make_async_copy(wf32.at[0], deg_sh.at[ed[0].at[1]], dsem).wait()
    plsc.subcore_barrier()

    pltpu.sync_copy(deg_sh, dinv)

    def rsqrt_step(i, carry):
        xv = dinv[pl.ds(i * _L, _L)]
        ib = lax.bitcast_convert_type(xv, jnp.int32)
        y = lax.bitcast_convert_type(jnp.int32(0x5F3759DF) - (ib >> 1),
                                     jnp.float32)
        y = y * (1.5 - 0.5 * xv * y * y)
        y = y * (1.5 - 0.5 * xv * y * y)
        y = y * (1.5 - 0.5 * xv * y * y)
        dinv[pl.ds(i * _L, _L)] = y
        return carry

    lax.fori_loop(0, n // _L, rsqrt_step, 0)

    start_echunk(c * half, 0)

    def main_group(gq, carry):
        for slot in range(2):
            i = gq * 2 + slot
            wait_echunk(slot)

            gcp = pltpu.async_copy(q_hbm.at[ed[slot].at[0]], rows[slot],
                                   rsem)

            for j in range(_CH // _L):
                sv = ed[slot][0, pl.ds(j * _L, _L)]
                dv = ed[slot][1, pl.ds(j * _L, _L)]
                wv = lax.bitcast_convert_type(
                    ed[slot][2, pl.ds(j * _L, _L)], jnp.float32)
                nbuf[pl.ds(j * _L, _L)] = (plsc.load_gather(dinv, [sv]) * wv
                                           * plsc.load_gather(dinv, [dv]))

            @pl.when(i >= 1)
            def _():
                pltpu.make_async_copy(rows[1 - slot],
                                      agg_sh.at[ed[0].at[1]], ssem).wait()

            @pl.when(i + 1 < half)
            def _():
                start_echunk(c * half + i + 1, 1 - slot)

            gcp.wait()

            def scale_step(g2, cc):
                n16 = nbuf[pl.ds(g2 * _L, _L)]
                for k in range(_L):
                    ns = n16[k]
                    e = g2 * _L + k
                    for j in range(fw // _L):
                        rows[slot][e, pl.ds(j * _L, _L)] = (
                            rows[slot][e, pl.ds(j * _L, _L)] * ns)
                return cc

            lax.fori_loop(0, _CH // _L, scale_step, 0)

            pltpu.async_copy(rows[slot], agg_sh.at[ed[slot].at[1]],
                             ssem, add=True)
        return carry

    lax.fori_loop(0, half // 2, main_group, 0)
    pltpu.make_async_copy(rows[1], agg_sh.at[ed[0].at[1]], ssem).wait()
    plsc.subcore_barrier()

    pltpu.sync_copy(agg_sh.at[pl.ds(s * zrows, zrows)],
                    agg_hbm.at[c, pl.ds(s * zrows, zrows)])
    if zrem:
        @pl.when(s == _NS - 1)
        def _():
            pltpu.sync_copy(agg_sh.at[pl.ds(_NS * zrows, zrem)],
                            agg_hbm.at[c, pl.ds(_NS * zrows, zrem)])


def kernel(x, edge_index, edge_weight, Wcz, bcz, Wcr, bcr, Wch, bch,
           Wz, bz, Wr, br, Wh, bh, Wl, bl):
    n, _ = x.shape
    fh = Wcz.shape[1]
    e = edge_weight.shape[0]

    q = pl.pallas_call(
        _fold_matmul_body,
        out_shape=jax.ShapeDtypeStruct((n, 2 * fh), jnp.float32),
    )(x, Wcz, Wch, Wz, Wh)

    loop = jnp.arange(n, dtype=edge_index.dtype)
    src = jnp.concatenate([edge_index[0], loop])
    dst = jnp.concatenate([edge_index[1], loop])
    w = jnp.concatenate([edge_weight, jnp.ones((n,), edge_weight.dtype)])
    grp = _NS * _CH * 4
    ep = ((e + n + grp - 1) // grp) * grp
    pad = ep - (e + n)
    padv = jnp.arange(pad, dtype=src.dtype) % n
    src = jnp.concatenate([src, padv])
    dst = jnp.concatenate([dst, padv])
    w = jnp.concatenate([w, jnp.zeros((pad,), w.dtype)])
    chunks = ep // (_NS * _CH)
    edata = jnp.stack(
        [src.reshape(-1, _CH), dst.reshape(-1, _CH),
         lax.bitcast_convert_type(w, jnp.int32).reshape(-1, _CH)], axis=1
    ).reshape(_NS, chunks, 3, _CH)

    z2 = jnp.zeros((n, 2 * fh), jnp.float32)
    z1 = jnp.zeros((n,), jnp.float32)

    mesh = plsc.VectorSubcoreMesh(core_axis_name="c", subcore_axis_name="s")
    agg2 = pl.kernel(
        _sc_body,
        out_type=jax.ShapeDtypeStruct((_NC, n, 2 * fh), jnp.float32),
        mesh=mesh,
        compiler_params=pltpu.CompilerParams(needs_layout_passes=False),
        scratch_types=[
            pltpu.VMEM((3, _CH), jnp.int32),
            pltpu.VMEM((3, _CH), jnp.int32),
            pltpu.VMEM((2, _CH), jnp.float32),
            pltpu.VMEM((_CH,), jnp.float32),
            pltpu.VMEM((_CH, 2 * fh), jnp.float32),
            pltpu.VMEM((_CH, 2 * fh), jnp.float32),
            pltpu.VMEM((n,), jnp.float32),
            pltpu.VMEM_SHARED((n,), jnp.float32),
            pltpu.VMEM_SHARED((n, 2 * fh), jnp.float32),
            pltpu.SemaphoreType.DMA,
            pltpu.SemaphoreType.DMA,
            pltpu.SemaphoreType.DMA,
            pltpu.SemaphoreType.DMA,
        ],
    )(q, edata, z2, z1)

    return pl.pallas_call(
        _tail_body,
        out_shape=jax.ShapeDtypeStruct((n, Wl.shape[1]), jnp.float32),
    )(agg2, bcz, bch, Wz, bz, Wh, bh, Wl, bl)
